# speedup vs baseline: 13.5261x; 13.5261x over previous
import functools

import jax
import jax.numpy as jnp
from jax import lax
from jax.experimental import pallas as pl
from jax.experimental.pallas import tpu as pltpu
from jax.experimental.pallas import tpu_sc as plsc

_NC = 2
_NS = 16
_NW = _NC * _NS


def _sc_gather(table, idx, n_rows, d):
    per_w = n_rows // _NW
    ch = 40
    n_ch = per_w // ch
    assert per_w % ch == 0 and per_w % 8 == 0 and ch % 8 == 0

    mesh = plsc.VectorSubcoreMesh(core_axis_name="c", subcore_axis_name="s")

    @functools.partial(
        pl.kernel,
        mesh=mesh,
        out_type=jax.ShapeDtypeStruct((n_rows, d), jnp.float32),
        scratch_types=[
            pltpu.VMEM((ch,), jnp.int32),
            pltpu.VMEM((ch,), jnp.int32),
            pltpu.VMEM((ch, d), jnp.float32),
            pltpu.VMEM((ch, d), jnp.float32),
            pltpu.SemaphoreType.DMA,
            pltpu.SemaphoreType.DMA,
        ],
    )
    def gather_kernel(table_hbm, idx_hbm, out_hbm, idx0, idx1, buf0, buf1,
                      sem0, sem1):
        wid = lax.axis_index("s") * _NC + lax.axis_index("c")
        base = wid * per_w

        def body(i, carry):
            off = base + i * (2 * ch)
            pltpu.sync_copy(idx_hbm.at[pl.ds(off, ch)], idx0)
            pltpu.async_copy(table_hbm.at[idx0], buf0, sem0).wait()
            pltpu.sync_copy(buf0, out_hbm.at[pl.ds(off, ch)])
            pltpu.sync_copy(idx_hbm.at[pl.ds(off + ch, ch)], idx1)
            pltpu.async_copy(table_hbm.at[idx1], buf1, sem1).wait()
            pltpu.sync_copy(buf1, out_hbm.at[pl.ds(off + ch, ch)])
            return carry

        lax.fori_loop(0, n_ch // 2, body, 0)

    return gather_kernel(table, idx)


def _tc_scan(g, fin, batch, length, s):
    d = s * s
    o = fin.shape[1]

    def scan_kernel(fin_ref, g_ref, r_ref, sfin_ref, s_ref):
        l = pl.program_id(0)

        @pl.when(l == 0)
        def _():
            col = lax.broadcasted_iota(jnp.int32, (batch, s), 1)
            s_ref[...] = jnp.where(col == 0, 1.0, 0.0).astype(jnp.float32)

        st = s_ref[...]
        gl = g_ref[0]

        row = lax.broadcasted_iota(jnp.int32, (s, d), 0)
        colk = lax.broadcasted_iota(jnp.int32, (s, d), 1) // s
        e = jnp.where(row == colk, 1.0, 0.0).astype(jnp.float32)
        se = jax.lax.dot_general(
            st, e, (((1,), (0,)), ((), ())),
            precision=lax.Precision.HIGHEST,
            preferred_element_type=jnp.float32)

        w = gl * se

        w128 = w[:, 0:128]
        for c in range(1, d // 128):
            w128 = w128 + w[:, c * 128:(c + 1) * 128]
        s_new = w128[:, 0:s]
        for q in range(1, 128 // s):
            s_new = s_new + w128[:, q * s:(q + 1) * s]

        s_ref[...] = s_new
        r_ref[0] = jax.lax.dot_general(
            s_new, fin_ref[...], (((1,), (0,)), ((), ())),
            precision=lax.Precision.HIGHEST,
            preferred_element_type=jnp.float32)

        @pl.when(l == length - 1)
        def _():
            sfin_ref[...] = s_new

    return pl.pallas_call(
        scan_kernel,
        grid=(length,),
        in_specs=[
            pl.BlockSpec((s, o), lambda l: (0, 0)),
            pl.BlockSpec((1, batch, d), lambda l: (l, 0, 0)),
        ],
        out_specs=[
            pl.BlockSpec((1, batch, o), lambda l: (l, 0, 0)),
            pl.BlockSpec((batch, s), lambda l: (0, 0)),
        ],
        out_shape=[
            jax.ShapeDtypeStruct((length, batch, o), jnp.float32),
            jax.ShapeDtypeStruct((batch, s), jnp.float32),
        ],
        scratch_shapes=[pltpu.VMEM((batch, s), jnp.float32)],
        compiler_params=pltpu.CompilerParams(
            dimension_semantics=("arbitrary",)),
    )(fin, g)


def kernel(action_seq, trans_prob, fin_matrix):
    batch, length = action_seq.shape
    a, s, _ = trans_prob.shape
    d = s * s

    table = jnp.reshape(trans_prob, (a, d))
    idx = jnp.reshape(jnp.transpose(action_seq, (1, 0)),
                      (length * batch,)).astype(jnp.int32)

    g = _sc_gather(table, idx, length * batch, d)
    g = jnp.reshape(g, (length, batch, d))

    rewards_t, s_final = _tc_scan(g, fin_matrix, batch, length, s)
    rewards = jnp.transpose(rewards_t, (1, 0, 2))
    return rewards, s_final
